# baseline (speedup 1.0000x reference)
import functools

import jax
import jax.numpy as jnp
import numpy as np
from jax import lax
from jax.experimental import pallas as pl
from jax.experimental.pallas import tpu as pltpu
from jax.experimental.pallas import tpu_sc as plsc

_MAX_DEGREE = 64
_NUM_SAMPLES = 32

_SAMPLE_COLS = np.array(
    [35, 45, 31, 63, 7, 4, 29, 44, 16, 58, 37, 19, 61, 2, 34, 5,
     30, 42, 3, 39, 56, 22, 6, 54, 18, 10, 11, 53, 32, 15, 49, 50],
    dtype=np.int32,
)


@functools.cache
def _build(B: int, N: int, D: int, S: int):
    info = plsc.get_sparse_core_info()
    nw = info.num_cores * info.num_subcores
    b_per_w = B // nw
    mesh = plsc.VectorSubcoreMesh(core_axis_name="c", subcore_axis_name="s")

    @functools.partial(
        pl.kernel,
        out_type=jax.ShapeDtypeStruct((B * S,), jnp.int32),
        mesh=mesh,
        compiler_params=pltpu.CompilerParams(
            needs_layout_passes=False,
            use_tc_tiling_on_sc=False,
            disable_bounds_checks=True,
        ),
        scratch_types=[
            pltpu.VMEM((b_per_w,), jnp.int32),
            pltpu.VMEM((b_per_w, D), jnp.int32),
            pltpu.VMEM((b_per_w * S,), jnp.int32),
            pltpu.VMEM((S,), jnp.int32),
            pltpu.SemaphoreType.DMA,
        ],
    )
    def k(adj_hbm, ids_hbm, cols_hbm, out_hbm, idx_v, rows_v, out_v, cols_v, sem):
        wid = lax.axis_index("s") * info.num_cores + lax.axis_index("c")
        base = wid * b_per_w
        pltpu.sync_copy(cols_hbm, cols_v)
        pltpu.sync_copy(ids_hbm.at[pl.ds(base, b_per_w)], idx_v)
        pltpu.async_copy(adj_hbm.at[idx_v], rows_v, sem).wait()
        c0 = cols_v[pl.ds(0, 16)]
        c1 = cols_v[pl.ds(16, 16)]

        def body(b, carry):
            out_v[pl.ds(b * S, 16)] = plsc.load_gather(rows_v.at[b], [c0])
            out_v[pl.ds(b * S + 16, 16)] = plsc.load_gather(rows_v.at[b], [c1])
            return carry

        lax.fori_loop(0, b_per_w, body, 0, unroll=8)
        pltpu.sync_copy(out_v, out_hbm.at[pl.ds(base * S, b_per_w * S)])

    return k


def kernel(adj_info, ids, num_samples):
    del num_samples
    B = ids.shape[0]
    N, D = adj_info.shape
    cols = jnp.asarray(_SAMPLE_COLS)
    flat = _build(B, N, D, _NUM_SAMPLES)(adj_info, ids, cols)
    return flat.reshape(B, _NUM_SAMPLES)


# speedup vs baseline: 1.3054x; 1.3054x over previous
import functools

import jax
import jax.numpy as jnp
import numpy as np
from jax import lax
from jax.experimental import pallas as pl
from jax.experimental.pallas import tpu as pltpu
from jax.experimental.pallas import tpu_sc as plsc

_MAX_DEGREE = 64
_NUM_SAMPLES = 32

_SAMPLE_COLS = np.array(
    [35, 45, 31, 63, 7, 4, 29, 44, 16, 58, 37, 19, 61, 2, 34, 5,
     30, 42, 3, 39, 56, 22, 6, 54, 18, 10, 11, 53, 32, 15, 49, 50],
    dtype=np.int32,
)


@functools.cache
def _build(B: int, N: int, D: int, S: int):
    info = plsc.get_sparse_core_info()
    nw = info.num_cores * info.num_subcores
    b_per_w = B // nw
    mesh = plsc.VectorSubcoreMesh(core_axis_name="c", subcore_axis_name="s")

    @functools.partial(
        pl.kernel,
        out_type=jax.ShapeDtypeStruct((B * S,), jnp.int32),
        mesh=mesh,
        compiler_params=pltpu.CompilerParams(
            needs_layout_passes=False,
            disable_bounds_checks=True,
        ),
        scratch_types=[
            pltpu.VMEM((b_per_w,), jnp.int32),
            pltpu.VMEM((b_per_w, D), jnp.int32),
            pltpu.VMEM((b_per_w * S,), jnp.int32),
            pltpu.VMEM((S,), jnp.int32),
            pltpu.SemaphoreType.DMA,
        ],
    )
    def k(adj_hbm, ids_hbm, cols_hbm, out_hbm, idx_v, rows_v, out_v, cols_v, sem):
        wid = lax.axis_index("s") * info.num_cores + lax.axis_index("c")
        base = wid * b_per_w
        pltpu.sync_copy(cols_hbm, cols_v)
        pltpu.sync_copy(ids_hbm.at[pl.ds(base, b_per_w)], idx_v)

        def fire(g, carry):
            vec = idx_v[pl.ds(g * 16, 16)]
            for j in range(16):
                pltpu.async_copy(adj_hbm.at[vec[j]], rows_v.at[g * 16 + j], sem)
            return carry

        lax.fori_loop(0, b_per_w // 16, fire, 0)

        def drain(i, carry):
            pltpu.make_async_copy(adj_hbm.at[0], rows_v.at[i], sem).wait()
            return carry

        lax.fori_loop(0, b_per_w, drain, 0)

        c0 = cols_v[pl.ds(0, 16)]
        c1 = cols_v[pl.ds(16, 16)]

        def body(b, carry):
            out_v[pl.ds(b * S, 16)] = plsc.load_gather(rows_v.at[b], [c0])
            out_v[pl.ds(b * S + 16, 16)] = plsc.load_gather(rows_v.at[b], [c1])
            return carry

        lax.fori_loop(0, b_per_w, body, 0, unroll=8)
        pltpu.sync_copy(out_v, out_hbm.at[pl.ds(base * S, b_per_w * S)])

    return k


def kernel(adj_info, ids, num_samples):
    del num_samples
    B = ids.shape[0]
    N, D = adj_info.shape
    cols = jnp.asarray(_SAMPLE_COLS)
    flat = _build(B, N, D, _NUM_SAMPLES)(adj_info, ids, cols)
    return flat.reshape(B, _NUM_SAMPLES)
